# baseline (speedup 1.0000x reference)
import functools

import jax
import jax.numpy as jnp
from jax import lax
from jax.experimental import pallas as pl
from jax.experimental.pallas import tpu as pltpu
from jax.experimental.pallas import tpu_sc as plsc

_N = 10000
_E = 320000
_D = 128
_H = 256
_NC = 2
_NS = 16
_CHUNK = 64
_NBUF = 4
_IBLK = 40
_E_PAD = 327680
_NCHUNKS = _E_PAD // _CHUNK
_N_OUT = 10240
_RPT = _N_OUT // _NS
_ZROWS = 32


def _make_sc_agg(num_tables: int, split_edges: bool):
    epc = _E_PAD // _NC if split_edges else _E_PAD
    ept = epc // _NS
    cpt = ept // _CHUNK
    nfull = cpt // _IBLK
    rem_blk = cpt % _IBLK
    assert nfull >= 1 and _IBLK % _NBUF == 0

    mesh = plsc.VectorSubcoreMesh(core_axis_name="c", subcore_axis_name="s")
    scratch = [
        pltpu.VMEM((_IBLK, _CHUNK), jnp.int32),
        pltpu.VMEM((_IBLK, _CHUNK), jnp.int32),
        [pltpu.VMEM((_CHUNK, _D), jnp.float32) for _ in range(_NBUF)],
        pltpu.VMEM((_ZROWS, _D), jnp.float32),
        pltpu.VMEM_SHARED((_N_OUT, _D), jnp.float32),
        pltpu.SemaphoreType.DMA,
        pltpu.SemaphoreType.DMA,
    ]

    @functools.partial(
        pl.kernel,
        out_type=jax.ShapeDtypeStruct((_NC, _N_OUT, _D), jnp.float32),
        mesh=mesh,
        scratch_types=scratch,
    )
    def sc_agg(*refs):
        if num_tables == 1:
            t0, eidx, outr, src_i, dst_i, rows, zbuf, acc, sem_g, sem_s = refs
            t1 = t0
        else:
            t0, t1, eidx, outr, src_i, dst_i, rows, zbuf, acc, sem_g, sem_s = refs
        c = lax.axis_index("c")
        s = lax.axis_index("s")

        cb = (c * (epc // _CHUNK) if split_edges else 0) + s * cpt

        def fire_gather(k, buf):
            if num_tables == 1:
                pltpu.async_copy(t0.at[src_i.at[k]], buf, sem_g)
            else:
                @pl.when(c == 0)
                def _g0():
                    pltpu.async_copy(t0.at[src_i.at[k]], buf, sem_g)

                @pl.when(c == 1)
                def _g1():
                    pltpu.async_copy(t1.at[src_i.at[k]], buf, sem_g)

        def wait_gather(k, buf):
            pltpu.make_async_copy(t0.at[src_i.at[k]], buf, sem_g).wait()

        def fire_scatter(k, buf):
            pltpu.async_copy(buf, acc.at[dst_i.at[k]], sem_s, add=True)

        def wait_scatter(k, buf):
            pltpu.make_async_copy(buf, acc.at[dst_i.at[k]], sem_s).wait()

        def stage(base_chunk, nchunks):
            pltpu.sync_copy(eidx.at[0, pl.ds(base_chunk, nchunks)],
                            src_i.at[pl.ds(0, nchunks)])
            pltpu.sync_copy(eidx.at[1, pl.ds(base_chunk, nchunks)],
                            dst_i.at[pl.ds(0, nchunks)])

        def prologue():
            for b in range(_NBUF - 1):
                fire_gather(b, rows[b])

        def pipe(nchunks):
            ngrp = nchunks // _NBUF

            def grp(g, carry2):
                for b in range(_NBUF):
                    k = g * _NBUF + b
                    wait_gather(k, rows[b])
                    fire_scatter(k, rows[b])
                    prev = rows[(b - 1) % _NBUF]
                    nxt = rows[(b + _NBUF - 1) % _NBUF]
                    if b == 0:
                        @pl.when(g > 0)
                        def _ws():
                            wait_scatter(k - 1, prev)
                    else:
                        wait_scatter(k - 1, prev)
                    if _NBUF * (ngrp - 1) + b + _NBUF - 1 < nchunks:
                        fire_gather(k + _NBUF - 1, nxt)
                    else:
                        @pl.when(g < ngrp - 1)
                        def _fg():
                            fire_gather(k + _NBUF - 1, nxt)
                return carry2
            lax.fori_loop(0, ngrp, grp, 0)
            for k in range(ngrp * _NBUF, nchunks):
                b = k % _NBUF
                wait_gather(k, rows[b])
                fire_scatter(k, rows[b])
                wait_scatter(k - 1, rows[(b - 1) % _NBUF])
                if k + _NBUF - 1 < nchunks:
                    fire_gather(k + _NBUF - 1, rows[(b + _NBUF - 1) % _NBUF])
            wait_scatter(nchunks - 1, rows[(nchunks - 1) % _NBUF])

        stage(cb, _IBLK)
        prologue()

        def zrow(i, carry):
            zbuf[i // 8, pl.ds((i % 8) * 16, 16)] = jnp.zeros((16,), jnp.float32)
            return carry
        lax.fori_loop(0, _ZROWS * 8, zrow, 0)

        def zdma(j, carry):
            pltpu.sync_copy(zbuf, acc.at[pl.ds(s * _RPT + j * _ZROWS, _ZROWS)])
            return carry
        lax.fori_loop(0, _RPT // _ZROWS, zdma, 0)
        plsc.subcore_barrier()

        pipe(_IBLK)

        def blk(j, carry):
            stage(cb + j * _IBLK, _IBLK)
            prologue()
            pipe(_IBLK)
            return carry
        if nfull > 1:
            lax.fori_loop(1, nfull, blk, 0)
        if rem_blk:
            stage(cb + nfull * _IBLK, rem_blk)
            prologue()
            pipe(rem_blk)
        plsc.subcore_barrier()

        pltpu.sync_copy(acc.at[pl.ds(s * _RPT, _RPT)],
                        outr.at[c, pl.ds(s * _RPT, _RPT)])

    return sc_agg


_sc_agg_conv1 = _make_sc_agg(num_tables=1, split_edges=True)
_sc_agg_conv2 = _make_sc_agg(num_tables=2, split_edges=False)

_BM = 1000


def _split_bf16(w):
    hi = w.astype(jnp.bfloat16)
    lo = (w - hi.astype(jnp.float32)).astype(jnp.bfloat16)
    return hi, lo


def _dot3(a, w_hi, w_lo):
    a_hi = a.astype(jnp.bfloat16)
    a_lo = (a - a_hi.astype(jnp.float32)).astype(jnp.bfloat16)
    f32 = jnp.float32
    return (jnp.dot(a_hi, w_hi, preferred_element_type=f32)
            + jnp.dot(a_hi, w_lo, preferred_element_type=f32)
            + jnp.dot(a_lo, w_hi, preferred_element_type=f32))


def _mlp1_body(x_ref, agg_ref0, agg_ref1, wah, wal, ba, wbh, wbl, bb, h0o, h1o):
    h = x_ref[...] + agg_ref0[0] + agg_ref1[0]
    a = _dot3(h, wah[...], wal[...]) + ba[...]
    a = jnp.maximum(a, 0.0)
    m = _dot3(a, wbh[...], wbl[...]) + bb[...]
    m = jnp.maximum(m, 0.0)
    h0o[...] = m[:, :_D]
    h1o[...] = m[:, _D:]


_mlp1_call = pl.pallas_call(
    _mlp1_body,
    grid=(_N // _BM,),
    in_specs=[
        pl.BlockSpec((_BM, _D), lambda i: (i, 0)),
        pl.BlockSpec((1, _BM, _D), lambda i: (0, i, 0)),
        pl.BlockSpec((1, _BM, _D), lambda i: (1, i, 0)),
        pl.BlockSpec((_D, _H), lambda i: (0, 0)),
        pl.BlockSpec((_D, _H), lambda i: (0, 0)),
        pl.BlockSpec((1, _H), lambda i: (0, 0)),
        pl.BlockSpec((_H, _H), lambda i: (0, 0)),
        pl.BlockSpec((_H, _H), lambda i: (0, 0)),
        pl.BlockSpec((1, _H), lambda i: (0, 0)),
    ],
    out_specs=[pl.BlockSpec((_BM, _D), lambda i: (i, 0))] * 2,
    out_shape=[jax.ShapeDtypeStruct((_N, _D), jnp.float32)] * 2,
)


def _mlp2_body(h0_ref, h1_ref, agg_ref0, agg_ref1, wah, wal, ba, wbh, wbl, bb,
               wl, blr, outo):
    h = jnp.concatenate([h0_ref[...] + agg_ref0[0],
                         h1_ref[...] + agg_ref1[0]], axis=1)
    a = _dot3(h, wah[...], wal[...]) + ba[...]
    a = jnp.maximum(a, 0.0)
    m = _dot3(a, wbh[...], wbl[...]) + bb[...]
    m = jnp.maximum(m, 0.0)
    outo[...] = jnp.dot(m, wl[...], preferred_element_type=jnp.float32) + blr[...]


_mlp2_call = pl.pallas_call(
    _mlp2_body,
    grid=(_N // _BM,),
    in_specs=[
        pl.BlockSpec((_BM, _D), lambda i: (i, 0)),
        pl.BlockSpec((_BM, _D), lambda i: (i, 0)),
        pl.BlockSpec((1, _BM, _D), lambda i: (0, i, 0)),
        pl.BlockSpec((1, _BM, _D), lambda i: (1, i, 0)),
        pl.BlockSpec((_H, _H), lambda i: (0, 0)),
        pl.BlockSpec((_H, _H), lambda i: (0, 0)),
        pl.BlockSpec((1, _H), lambda i: (0, 0)),
        pl.BlockSpec((_H, _H), lambda i: (0, 0)),
        pl.BlockSpec((_H, _H), lambda i: (0, 0)),
        pl.BlockSpec((1, _H), lambda i: (0, 0)),
        pl.BlockSpec((_H, 2), lambda i: (0, 0)),
        pl.BlockSpec((1, 2), lambda i: (0, 0)),
    ],
    out_specs=pl.BlockSpec((_BM, 2), lambda i: (i, 0)),
    out_shape=jax.ShapeDtypeStruct((_N, 2), jnp.float32),
)


def kernel(x, edge_index, W1a, b1a, W1b, b1b, W2a, b2a, W2b, b2b, Wl, bl):
    ei = edge_index.astype(jnp.int32)
    pad = _E_PAD - _E
    r = jnp.arange(pad, dtype=jnp.int32)
    pad_cols = jnp.stack([r % _N, _N + r % (_N_OUT - _N)])
    eidx = jnp.concatenate([ei, pad_cols], axis=1).reshape(2, _NCHUNKS, _CHUNK)

    w1ah, w1al = _split_bf16(W1a)
    w1bh, w1bl = _split_bf16(W1b)
    w2ah, w2al = _split_bf16(W2a)
    w2bh, w2bl = _split_bf16(W2b)

    agg1 = _sc_agg_conv1(x, eidx)
    h0, h1 = _mlp1_call(x, agg1, agg1,
                        w1ah, w1al, b1a.reshape(1, _H),
                        w1bh, w1bl, b1b.reshape(1, _H))
    agg2 = _sc_agg_conv2(h0, h1, eidx)
    out = _mlp2_call(h0, h1, agg2, agg2,
                     w2ah, w2al, b2a.reshape(1, _H),
                     w2bh, w2bl, b2b.reshape(1, _H),
                     Wl, bl.reshape(1, 2))
    return out


# speedup vs baseline: 1.0326x; 1.0326x over previous
import functools

import jax
import jax.numpy as jnp
from jax import lax
from jax.experimental import pallas as pl
from jax.experimental.pallas import tpu as pltpu
from jax.experimental.pallas import tpu_sc as plsc

_N = 10000
_E = 320000
_D = 128
_H = 256
_NC = 2
_NS = 16
_CHUNK = 64
_NBUF = 4
_IBLK = 40
_E_PAD = 327680
_NCHUNKS = _E_PAD // _CHUNK
_N_OUT = 10240
_RPT = _N_OUT // _NS
_ZROWS = 32


def _make_sc_agg(num_tables: int, split_edges: bool):
    epc = _E_PAD // _NC if split_edges else _E_PAD
    ept = epc // _NS
    cpt = ept // _CHUNK
    nfull = cpt // _IBLK
    rem_blk = cpt % _IBLK
    assert nfull >= 1 and _IBLK % _NBUF == 0

    mesh = plsc.VectorSubcoreMesh(core_axis_name="c", subcore_axis_name="s")
    scratch = [
        pltpu.VMEM((_IBLK, _CHUNK), jnp.int32),
        pltpu.VMEM((_IBLK, _CHUNK), jnp.int32),
        [pltpu.VMEM((_CHUNK, _D), jnp.float32) for _ in range(_NBUF)],
        pltpu.VMEM((_ZROWS, _D), jnp.float32),
        pltpu.VMEM_SHARED((_N_OUT, _D), jnp.float32),
        pltpu.SemaphoreType.DMA,
        pltpu.SemaphoreType.DMA,
    ]

    @functools.partial(
        pl.kernel,
        out_type=jax.ShapeDtypeStruct((_NC, _N_OUT, _D), jnp.float32),
        mesh=mesh,
        scratch_types=scratch,
    )
    def sc_agg(*refs):
        if num_tables == 1:
            t0, eidx, outr, src_i, dst_i, rows, zbuf, acc, sem_g, sem_s = refs
            t1 = t0
        else:
            t0, t1, eidx, outr, src_i, dst_i, rows, zbuf, acc, sem_g, sem_s = refs
        c = lax.axis_index("c")
        s = lax.axis_index("s")

        cb = (c * (epc // _CHUNK) if split_edges else 0) + s * cpt

        def fire_gather(k, buf):
            if num_tables == 1:
                pltpu.async_copy(t0.at[src_i.at[k]], buf, sem_g)
            else:
                @pl.when(c == 0)
                def _g0():
                    pltpu.async_copy(t0.at[src_i.at[k]], buf, sem_g)

                @pl.when(c == 1)
                def _g1():
                    pltpu.async_copy(t1.at[src_i.at[k]], buf, sem_g)

        def wait_gather(k, buf):
            pltpu.make_async_copy(t0.at[src_i.at[k]], buf, sem_g).wait()

        def fire_scatter(k, buf):
            pltpu.async_copy(buf, acc.at[dst_i.at[k]], sem_s, add=True)

        def wait_scatter(k, buf):
            pltpu.make_async_copy(buf, acc.at[dst_i.at[k]], sem_s).wait()

        def stage(base_chunk, nchunks):
            pltpu.sync_copy(eidx.at[0, pl.ds(base_chunk, nchunks)],
                            src_i.at[pl.ds(0, nchunks)])
            pltpu.sync_copy(eidx.at[1, pl.ds(base_chunk, nchunks)],
                            dst_i.at[pl.ds(0, nchunks)])

        def prologue():
            for b in range(_NBUF - 1):
                fire_gather(b, rows[b])

        def pipe(nchunks):
            ngrp = nchunks // _NBUF

            def grp(g, carry2):
                for b in range(_NBUF):
                    k = g * _NBUF + b
                    wait_gather(k, rows[b])
                    fire_scatter(k, rows[b])
                    prev = rows[(b - 1) % _NBUF]
                    nxt = rows[(b + _NBUF - 1) % _NBUF]
                    if b == 0:
                        @pl.when(g > 0)
                        def _ws():
                            wait_scatter(k - 1, prev)
                    else:
                        wait_scatter(k - 1, prev)
                    if _NBUF * (ngrp - 1) + b + _NBUF - 1 < nchunks:
                        fire_gather(k + _NBUF - 1, nxt)
                    else:
                        @pl.when(g < ngrp - 1)
                        def _fg():
                            fire_gather(k + _NBUF - 1, nxt)
                return carry2
            lax.fori_loop(0, ngrp, grp, 0)
            for k in range(ngrp * _NBUF, nchunks):
                b = k % _NBUF
                wait_gather(k, rows[b])
                fire_scatter(k, rows[b])
                wait_scatter(k - 1, rows[(b - 1) % _NBUF])
                if k + _NBUF - 1 < nchunks:
                    fire_gather(k + _NBUF - 1, rows[(b + _NBUF - 1) % _NBUF])
            wait_scatter(nchunks - 1, rows[(nchunks - 1) % _NBUF])

        stage(cb, _IBLK)
        prologue()

        def zrow(i, carry):
            zbuf[i // 8, pl.ds((i % 8) * 16, 16)] = jnp.zeros((16,), jnp.float32)
            return carry
        lax.fori_loop(0, _ZROWS * 8, zrow, 0)

        def zdma(j, carry):
            pltpu.sync_copy(zbuf, acc.at[pl.ds(s * _RPT + j * _ZROWS, _ZROWS)])
            return carry
        lax.fori_loop(0, _RPT // _ZROWS, zdma, 0)
        plsc.subcore_barrier()

        pipe(_IBLK)

        def blk(j, carry):
            stage(cb + j * _IBLK, _IBLK)
            prologue()
            pipe(_IBLK)
            return carry
        if nfull > 1:
            lax.fori_loop(1, nfull, blk, 0)
        if rem_blk:
            stage(cb + nfull * _IBLK, rem_blk)
            prologue()
            pipe(rem_blk)
        plsc.subcore_barrier()

        pltpu.sync_copy(acc.at[pl.ds(s * _RPT, _RPT)],
                        outr.at[c, pl.ds(s * _RPT, _RPT)])

    return sc_agg


_sc_agg_conv1 = _make_sc_agg(num_tables=1, split_edges=True)
_sc_agg_conv2 = _make_sc_agg(num_tables=2, split_edges=False)

_BM = 1000


def _mlp1_body(x_ref, agg_ref0, agg_ref1, wa, ba, wb, bb, h0o, h1o):
    h = x_ref[...] + agg_ref0[0] + agg_ref1[0]
    a = jnp.dot(h, wa[...], preferred_element_type=jnp.float32) + ba[...]
    a = jnp.maximum(a, 0.0)
    m = jnp.dot(a, wb[...], preferred_element_type=jnp.float32) + bb[...]
    m = jnp.maximum(m, 0.0)
    h0o[...] = m[:, :_D]
    h1o[...] = m[:, _D:]


_mlp1_call = pl.pallas_call(
    _mlp1_body,
    grid=(_N // _BM,),
    in_specs=[
        pl.BlockSpec((_BM, _D), lambda i: (i, 0)),
        pl.BlockSpec((1, _BM, _D), lambda i: (0, i, 0)),
        pl.BlockSpec((1, _BM, _D), lambda i: (1, i, 0)),
        pl.BlockSpec((_D, _H), lambda i: (0, 0)),
        pl.BlockSpec((1, _H), lambda i: (0, 0)),
        pl.BlockSpec((_H, _H), lambda i: (0, 0)),
        pl.BlockSpec((1, _H), lambda i: (0, 0)),
    ],
    out_specs=[pl.BlockSpec((_BM, _D), lambda i: (i, 0))] * 2,
    out_shape=[jax.ShapeDtypeStruct((_N, _D), jnp.float32)] * 2,
)


def _mlp2_body(h0_ref, h1_ref, agg_ref0, agg_ref1, wa, ba, wb, bb, wl, blr,
               outo):
    h = jnp.concatenate([h0_ref[...] + agg_ref0[0],
                         h1_ref[...] + agg_ref1[0]], axis=1)
    a = jnp.dot(h, wa[...], preferred_element_type=jnp.float32) + ba[...]
    a = jnp.maximum(a, 0.0)
    m = jnp.dot(a, wb[...], preferred_element_type=jnp.float32) + bb[...]
    m = jnp.maximum(m, 0.0)
    outo[...] = jnp.dot(m, wl[...], preferred_element_type=jnp.float32) + blr[...]


_mlp2_call = pl.pallas_call(
    _mlp2_body,
    grid=(_N // _BM,),
    in_specs=[
        pl.BlockSpec((_BM, _D), lambda i: (i, 0)),
        pl.BlockSpec((_BM, _D), lambda i: (i, 0)),
        pl.BlockSpec((1, _BM, _D), lambda i: (0, i, 0)),
        pl.BlockSpec((1, _BM, _D), lambda i: (1, i, 0)),
        pl.BlockSpec((_H, _H), lambda i: (0, 0)),
        pl.BlockSpec((1, _H), lambda i: (0, 0)),
        pl.BlockSpec((_H, _H), lambda i: (0, 0)),
        pl.BlockSpec((1, _H), lambda i: (0, 0)),
        pl.BlockSpec((_H, 2), lambda i: (0, 0)),
        pl.BlockSpec((1, 2), lambda i: (0, 0)),
    ],
    out_specs=pl.BlockSpec((_BM, 2), lambda i: (i, 0)),
    out_shape=jax.ShapeDtypeStruct((_N, 2), jnp.float32),
)


def kernel(x, edge_index, W1a, b1a, W1b, b1b, W2a, b2a, W2b, b2b, Wl, bl):
    ei = edge_index.astype(jnp.int32)
    pad = _E_PAD - _E
    r = jnp.arange(pad, dtype=jnp.int32)
    pad_cols = jnp.stack([r % _N, _N + r % (_N_OUT - _N)])
    eidx = jnp.concatenate([ei, pad_cols], axis=1).reshape(2, _NCHUNKS, _CHUNK)

    agg1 = _sc_agg_conv1(x, eidx)
    h0, h1 = _mlp1_call(x, agg1, agg1,
                        W1a, b1a.reshape(1, _H), W1b, b1b.reshape(1, _H))
    agg2 = _sc_agg_conv2(h0, h1, eidx)
    out = _mlp2_call(h0, h1, agg2, agg2,
                     W2a, b2a.reshape(1, _H), W2b, b2b.reshape(1, _H),
                     Wl, bl.reshape(1, 2))
    return out
